# baseline (speedup 1.0000x reference)
import functools

import jax
import jax.numpy as jnp
from jax import lax
from jax.experimental import pallas as pl
from jax.experimental.pallas import tpu as pltpu
from jax.experimental.pallas import tpu_sc as plsc

HIGHEST = jax.lax.Precision.HIGHEST
F32 = jnp.float32
I32 = jnp.int32


def _mm(a, b, dims, prec=jax.lax.Precision.DEFAULT):
    return jax.lax.dot_general(a, b, (dims, ((), ())), precision=prec)


def _sel_kernel(xt_ref, bnd_ref, w1_ref, b1_ref, w2_ref, b2_ref,
                imp_ref, selm_ref, bounds_ref, *, N, K):
    h1t = _mm(w1_ref[...], xt_ref[0], ((1,), (0,)), HIGHEST) + b1_ref[...]
    h1t = h1t * F32(0.5) * (F32(1.0) + jax.lax.erf(h1t * F32(0.7071067811865476)))
    logit = _mm(w2_ref[...], h1t, ((1,), (0,)), HIGHEST) + b2_ref[...]
    imp = jax.nn.sigmoid(logit) + F32(0.5) * bnd_ref[0]
    imp_ref[0] = imp

    bits = jax.lax.bitcast_convert_type(imp, I32)

    def cnt_ge(thr):
        return jnp.sum((bits >= thr).astype(I32))

    def t_body(i, t):
        cand = t | (I32(1) << (I32(30) - i))
        return jnp.where(cnt_ge(cand) >= K, cand, t)

    t = jax.lax.fori_loop(0, 31, t_body, I32(0))
    need = K - cnt_ge(t + 1)

    idx_row = jax.lax.broadcasted_iota(I32, (1, N), 1)
    key = jnp.where(bits == t, I32(N - 1) - idx_row, I32(-1))

    def th_body(i, th):
        cand = th | (I32(1) << (I32(11) - i))
        return jnp.where(jnp.sum((key >= cand).astype(I32)) >= need, cand, th)

    th2 = jax.lax.fori_loop(0, 12, th_body, I32(0))
    sel_row = ((bits > t) | (key >= th2)).astype(F32)
    selm_ref[0] = sel_row

    sel32 = sel_row.reshape(N // 128, 128)
    NT2 = N // 128
    lane_i = jax.lax.broadcasted_iota(I32, (128, 128), 0)
    lane_j = jax.lax.broadcasted_iota(I32, (128, 128), 1)
    Ustrict = (lane_i < lane_j).astype(F32)
    prefix_in = _mm(sel32, Ustrict, ((1,), (0,)))
    rowsum = jnp.sum(sel32, axis=1, keepdims=True)
    row_i = jax.lax.broadcasted_iota(I32, (NT2, NT2), 0)
    row_j = jax.lax.broadcasted_iota(I32, (NT2, NT2), 1)
    Lstrict = (row_j < row_i).astype(F32)
    offs = _mm(Lstrict, rowsum, ((1,), (0,)))
    bounds_ref[0] = jnp.concatenate(
        [offs, offs + prefix_in[:, 64:65]], axis=1)


def _attn_kernel(xs_ref, wq_ref, bq_ref, wk_ref, bk_ref, wv_ref, bv_ref,
                 wo_ref, bo_ref, lng_ref, lnb_ref,
                 enh_ref, q_ref, k_ref, v_ref, ctx_ref, *, K, heads, hd):
    C = wq_ref.shape[1]
    scale = F32(hd) ** -0.5

    def lin(a, w_ref, b_ref):
        return _mm(a, w_ref[...], ((1,), (1,))) + b_ref[...]

    xs = xs_ref[0, :, 0:C]
    q_ref[...] = lin(xs, wq_ref, bq_ref)
    k_ref[...] = lin(xs, wk_ref, bk_ref)
    v_ref[...] = lin(xs, wv_ref, bv_ref)
    ctx_ref[...] = jnp.zeros((K, C), dtype=F32)
    col = jax.lax.broadcasted_iota(I32, (1, C), 1)

    def head_body(h, carry):
        hm = ((col >= h * hd) & (col < (h + 1) * hd)).astype(F32)
        e = jnp.exp(_mm(q_ref[...] * hm, k_ref[...], ((1,), (1,))) * scale)
        ctx = _mm(e, v_ref[...] * hm, ((1,), (0,)))
        ctx_ref[...] += ctx / jnp.sum(e, axis=1, keepdims=True)
        return carry

    jax.lax.fori_loop(0, heads, head_body, 0)

    y = lin(ctx_ref[...], wo_ref, bo_ref) + xs
    mu = jnp.mean(y, axis=1, keepdims=True)
    var = jnp.mean((y - mu) ** 2, axis=1, keepdims=True)
    enh = (y - mu) * jax.lax.rsqrt(var + F32(1e-5)) * lng_ref[...] + lnb_ref[...]
    CP = enh_ref.shape[2]
    enh_ref[0, :K, :] = jnp.concatenate(
        [enh, jnp.zeros((K, CP - C), F32)], axis=1)


def _make_sc_kernels(B, N, K, C, NT):
    CP = 128
    NC, NS = 2, 16
    NW = NC * NS
    SLOTS = K // NW
    TOK = N // NW
    NP = N + 8
    mesh = plsc.VectorSubcoreMesh(core_axis_name="c", subcore_axis_name="s")

    @functools.partial(
        pl.kernel, mesh=mesh,
        compiler_params=pltpu.CompilerParams(needs_layout_passes=False),
        out_type=jax.ShapeDtypeStruct((B * K,), I32),
        scratch_types=[pltpu.VMEM((N,), F32), pltpu.VMEM((K + 16,), I32)],
    )
    def compact(selm_hbm, idx_hbm, maskv, idxbuf):
        iota16 = lax.iota(I32, 16)
        wid = lax.axis_index("s") * NC + lax.axis_index("c")

        @pl.when(wid < B)
        def _():
            b = wid
            pltpu.sync_copy(selm_hbm.at[pl.ds(b * N, N)], maskv)

            def body(cvec, cursor):
                mi = maskv[pl.ds(cvec * 16, 16)].astype(I32)
                vals = iota16 + (cvec * 16 + b * N)
                pos = (mi * (cursor + plsc.cumsum(mi) - 1)
                       + (1 - mi) * I32(K + 15))
                plsc.store_scatter(idxbuf, [pos], vals)
                return cursor + jnp.sum(mi)

            jax.lax.fori_loop(0, N // 16, body, I32(0))
            pltpu.sync_copy(idxbuf.at[pl.ds(0, K)],
                            idx_hbm.at[pl.ds(b * K, K)])

    @functools.partial(
        pl.kernel, mesh=mesh,
        compiler_params=pltpu.CompilerParams(needs_layout_passes=False),
        out_type=jax.ShapeDtypeStruct((B * K, CP), F32),
        scratch_types=[pltpu.VMEM((SLOTS,), I32), pltpu.VMEM((SLOTS, CP), F32),
                       pltpu.SemaphoreType.DMA],
    )
    def gather(x_hbm, idx_hbm, xs_hbm, idx_v, rows_v, sem):
        wid = lax.axis_index("s") * NC + lax.axis_index("c")
        for b in range(B):
            base = b * K + wid * SLOTS
            pltpu.sync_copy(idx_hbm.at[pl.ds(base, SLOTS)], idx_v)
            pltpu.async_copy(x_hbm.at[idx_v], rows_v, sem).wait()
            pltpu.sync_copy(rows_v, xs_hbm.at[pl.ds(base, SLOTS)])

    RT = 64
    NR = N // RT
    WR = RT + 16
    ENHR = K + N // 16

    @functools.partial(
        pl.kernel, mesh=mesh,
        compiler_params=pltpu.CompilerParams(needs_layout_passes=False),
        out_type=jax.ShapeDtypeStruct((B * NP, CP), F32),
        scratch_types=[pltpu.VMEM((RT,), F32),
                       pltpu.VMEM((NT,), F32),
                       pltpu.VMEM((WR,), I32),
                       pltpu.VMEM((RT, CP), F32),
                       pltpu.VMEM((WR, CP), F32),
                       pltpu.SemaphoreType.DMA],
    )
    def scatter(x_hbm, selm_hbm, bounds_hbm, enh_hbm, out_hbm,
                maskv, bv, destv, xbuf, ebuf, sem):
        iota16 = lax.iota(I32, 16)
        wid = lax.axis_index("s") * NC + lax.axis_index("c")
        for b in range(B):
            pltpu.sync_copy(bounds_hbm.at[pl.ds(b * NT, NT)], bv)
            for gi in range(NR // NW):
                g = wid + gi * NW
                tok0 = b * N + g * RT
                pltpu.sync_copy(x_hbm.at[pl.ds(tok0, RT), :], xbuf)
                pltpu.sync_copy(xbuf,
                                out_hbm.at[pl.ds(b * NP + g * RT, RT), :])
                lo = I32(0)
                for v in range(NT // 16):
                    hit = (1 - jnp.minimum(jnp.abs(iota16 + v * 16 - g), 1))
                    lo = lo + jnp.sum(bv[pl.ds(v * 16, 16)].astype(I32)
                                      * hit)
                lo_al = (lo // 8) * 8
                sh = lo - lo_al
                pltpu.sync_copy(selm_hbm.at[pl.ds(tok0, RT)], maskv)
                for cvec in range(WR // 16):
                    destv[pl.ds(cvec * 16, 16)] = jnp.full((16,), b * NP + N,
                                                           I32)

                def body(cvec, cursor):
                    mi = maskv[pl.ds(cvec * 16, 16)].astype(I32)
                    vals = (mi * (iota16 + (cvec * 16 + b * NP + g * RT))
                            + (1 - mi) * I32(b * NP + N))
                    pos = (mi * (sh + cursor + plsc.cumsum(mi) - 1)
                           + (1 - mi) * I32(WR - 1))
                    plsc.store_scatter(destv, [pos], vals)
                    return cursor + jnp.sum(mi)

                jax.lax.fori_loop(0, RT // 16, body, I32(0))
                pltpu.sync_copy(
                    enh_hbm.at[pl.ds(b * ENHR + lo_al, WR), :], ebuf)
                pltpu.async_copy(ebuf, out_hbm.at[destv], sem).wait()

    return compact, gather, scatter


def kernel(x, boundary_map, w_imp1, b_imp1, w_imp2, b_imp2,
           Wq, bq, Wk, bk, Wv, bv, Wo, bo, ln_g, ln_b):
    B, C, H, W = x.shape
    N = H * W
    K = max(int(N * 0.25), 1)
    heads = 8
    hd = C // heads
    NT = N // 64
    NP = N + 8

    x_t = x.reshape(B, C, N)
    CP = 128
    x2d = jnp.pad(jnp.transpose(x_t, (0, 2, 1)).reshape(B * N, C),
                  ((0, 0), (0, CP - C)))
    bnd = boundary_map.reshape(B, 1, N)

    full = lambda s: pl.BlockSpec(s, lambda b: (0,) * len(s))
    imp, selm, bounds = pl.pallas_call(
        functools.partial(_sel_kernel, N=N, K=K),
        grid=(B,),
        in_specs=[
            pl.BlockSpec((1, C, N), lambda b: (b, 0, 0)),
            pl.BlockSpec((1, 1, N), lambda b: (b, 0, 0)),
            full((C // 4, C)), full((C // 4, 1)),
            full((1, C // 4)), full((1, 1)),
        ],
        out_specs=[
            pl.BlockSpec((1, 1, N), lambda b: (b, 0, 0)),
            pl.BlockSpec((1, 1, N), lambda b: (b, 0, 0)),
            pl.BlockSpec((1, NT // 2, 2), lambda b: (b, 0, 0)),
        ],
        out_shape=[
            jax.ShapeDtypeStruct((B, 1, N), F32),
            jax.ShapeDtypeStruct((B, 1, N), F32),
            jax.ShapeDtypeStruct((B, NT // 2, 2), F32),
        ],
        compiler_params=pltpu.CompilerParams(
            dimension_semantics=("parallel",)),
    )(x_t, bnd, w_imp1, b_imp1.reshape(-1, 1), w_imp2, b_imp2.reshape(1, 1))

    compact, gather, scatter = _make_sc_kernels(B, N, K, C, NT)
    selm1d = selm.reshape(B * N)
    idx = compact(selm1d)
    xs2d = gather(x2d, idx)

    enh = pl.pallas_call(
        functools.partial(_attn_kernel, K=K, heads=heads, hd=hd),
        grid=(B,),
        in_specs=[
            pl.BlockSpec((1, K, CP), lambda b: (b, 0, 0)),
            full((C, C)), full((1, C)),
            full((C, C)), full((1, C)),
            full((C, C)), full((1, C)),
            full((C, C)), full((1, C)),
            full((1, C)), full((1, C)),
        ],
        out_specs=[pl.BlockSpec((1, K + N // 16, CP), lambda b: (b, 0, 0))],
        out_shape=[jax.ShapeDtypeStruct((B, K + N // 16, CP), F32)],
        compiler_params=pltpu.CompilerParams(
            dimension_semantics=("parallel",)),
        scratch_shapes=[pltpu.VMEM((K, C), F32)] * 4,
    )(xs2d.reshape(B, K, CP),
      Wq, bq.reshape(1, -1), Wk, bk.reshape(1, -1), Wv, bv.reshape(1, -1),
      Wo, bo.reshape(1, -1), ln_g.reshape(1, -1), ln_b.reshape(1, -1))[0]

    out2d = scatter(x2d, selm1d, bounds.reshape(B * NT),
                    enh.reshape(B * (K + N // 16), CP))
    out_flat = out2d.reshape(B, NP, CP)[:, :N, :C]
    out = jnp.transpose(out_flat.reshape(B, H, W, C), (0, 3, 1, 2))
    importance = imp.reshape(B, 1, H, W)
    return (out, importance)


# speedup vs baseline: 3.5154x; 3.5154x over previous
import functools

import jax
import jax.numpy as jnp
from jax import lax
from jax.experimental import pallas as pl
from jax.experimental.pallas import tpu as pltpu
from jax.experimental.pallas import tpu_sc as plsc

HIGHEST = jax.lax.Precision.HIGHEST
F32 = jnp.float32
I32 = jnp.int32


def _mm(a, b, dims, prec=jax.lax.Precision.DEFAULT):
    return jax.lax.dot_general(a, b, (dims, ((), ())), precision=prec)


def _sel_kernel(xt_ref, bnd_ref, w1_ref, b1_ref, w2_ref, b2_ref,
                imp_ref, selm_ref, bounds_ref, *, N, K):
    h1t = _mm(w1_ref[...], xt_ref[0], ((1,), (0,)), HIGHEST) + b1_ref[...]
    h1t = h1t * F32(0.5) * (F32(1.0) + jax.lax.erf(h1t * F32(0.7071067811865476)))
    logit = _mm(w2_ref[...], h1t, ((1,), (0,)), HIGHEST) + b2_ref[...]
    imp = jax.nn.sigmoid(logit) + F32(0.5) * bnd_ref[0]
    imp_ref[0] = imp

    bits = jax.lax.bitcast_convert_type(imp, I32)

    def cnt_ge(thr):
        return jnp.sum((bits >= thr).astype(I32))

    def t_body(i, t):
        cand = t | (I32(1) << (I32(30) - i))
        return jnp.where(cnt_ge(cand) >= K, cand, t)

    t = jax.lax.fori_loop(0, 31, t_body, I32(0))
    need = K - cnt_ge(t + 1)

    idx_row = jax.lax.broadcasted_iota(I32, (1, N), 1)
    key = jnp.where(bits == t, I32(N - 1) - idx_row, I32(-1))

    def th_body(i, th):
        cand = th | (I32(1) << (I32(11) - i))
        return jnp.where(jnp.sum((key >= cand).astype(I32)) >= need, cand, th)

    th2 = jax.lax.fori_loop(0, 12, th_body, I32(0))
    sel_row = ((bits > t) | (key >= th2)).astype(F32)
    selm_ref[0] = sel_row

    sel32 = sel_row.reshape(N // 128, 128)
    NT2 = N // 128
    lane_i = jax.lax.broadcasted_iota(I32, (128, 128), 0)
    lane_j = jax.lax.broadcasted_iota(I32, (128, 128), 1)
    Ustrict = (lane_i < lane_j).astype(F32)
    prefix_in = _mm(sel32, Ustrict, ((1,), (0,)))
    rowsum = jnp.sum(sel32, axis=1, keepdims=True)
    row_i = jax.lax.broadcasted_iota(I32, (NT2, NT2), 0)
    row_j = jax.lax.broadcasted_iota(I32, (NT2, NT2), 1)
    Lstrict = (row_j < row_i).astype(F32)
    offs = _mm(Lstrict, rowsum, ((1,), (0,)))
    bounds_ref[0] = jnp.concatenate(
        [offs, offs + prefix_in[:, 64:65]], axis=1)


def _attn_kernel(xs_ref, wq_ref, bq_ref, wk_ref, bk_ref, wv_ref, bv_ref,
                 wo_ref, bo_ref, lng_ref, lnb_ref,
                 enh_ref, q_ref, k_ref, v_ref, ctx_ref, *, K, heads, hd):
    C = wq_ref.shape[1]
    scale = F32(hd) ** -0.5

    def lin(a, w_ref, b_ref):
        return _mm(a, w_ref[...], ((1,), (1,))) + b_ref[...]

    xs = xs_ref[0, :, 0:C]
    q_ref[...] = lin(xs, wq_ref, bq_ref)
    k_ref[...] = lin(xs, wk_ref, bk_ref)
    v_ref[...] = lin(xs, wv_ref, bv_ref)
    ctx_ref[...] = jnp.zeros((K, C), dtype=F32)
    col = jax.lax.broadcasted_iota(I32, (1, C), 1)

    def head_body(h, carry):
        hm = ((col >= h * hd) & (col < (h + 1) * hd)).astype(F32)
        e = jnp.exp(_mm(q_ref[...] * hm, k_ref[...], ((1,), (1,))) * scale)
        ctx = _mm(e, v_ref[...] * hm, ((1,), (0,)))
        ctx_ref[...] += ctx / jnp.sum(e, axis=1, keepdims=True)
        return carry

    jax.lax.fori_loop(0, heads, head_body, 0)

    y = lin(ctx_ref[...], wo_ref, bo_ref) + xs
    mu = jnp.mean(y, axis=1, keepdims=True)
    var = jnp.mean((y - mu) ** 2, axis=1, keepdims=True)
    enh = (y - mu) * jax.lax.rsqrt(var + F32(1e-5)) * lng_ref[...] + lnb_ref[...]
    CP = enh_ref.shape[2]
    enh_ref[0, :K, :] = jnp.concatenate(
        [enh, jnp.zeros((K, CP - C), F32)], axis=1)


def _make_sc_kernels(B, N, K, C, NT):
    CP = 128
    NC, NS = 2, 16
    NW = NC * NS
    SLOTS = K // NW
    TOK = N // NW
    NP = N + 64
    mesh = plsc.VectorSubcoreMesh(core_axis_name="c", subcore_axis_name="s")

    @functools.partial(
        pl.kernel, mesh=mesh,
        compiler_params=pltpu.CompilerParams(needs_layout_passes=False),
        out_type=jax.ShapeDtypeStruct((B * K,), I32),
        scratch_types=[pltpu.VMEM((N,), F32), pltpu.VMEM((K + 16,), I32)],
    )
    def compact(selm_hbm, idx_hbm, maskv, idxbuf):
        iota16 = lax.iota(I32, 16)
        wid = lax.axis_index("s") * NC + lax.axis_index("c")

        @pl.when(wid < B)
        def _():
            b = wid
            pltpu.sync_copy(selm_hbm.at[pl.ds(b * N, N)], maskv)

            def body(cvec, cursor):
                mi = maskv[pl.ds(cvec * 16, 16)].astype(I32)
                vals = iota16 + (cvec * 16 + b * N)
                pos = (mi * (cursor + plsc.cumsum(mi) - 1)
                       + (1 - mi) * I32(K + 15))
                plsc.store_scatter(idxbuf, [pos], vals)
                return cursor + jnp.sum(mi)

            jax.lax.fori_loop(0, N // 16, body, I32(0))
            pltpu.sync_copy(idxbuf.at[pl.ds(0, K)],
                            idx_hbm.at[pl.ds(b * K, K)])

    @functools.partial(
        pl.kernel, mesh=mesh,
        compiler_params=pltpu.CompilerParams(needs_layout_passes=False),
        out_type=jax.ShapeDtypeStruct((B * K, CP), F32),
        scratch_types=[pltpu.VMEM((SLOTS,), I32), pltpu.VMEM((SLOTS, CP), F32),
                       pltpu.SemaphoreType.DMA],
    )
    def gather(x_hbm, idx_hbm, xs_hbm, idx_v, rows_v, sem):
        wid = lax.axis_index("s") * NC + lax.axis_index("c")
        for b in range(B):
            base = b * K + wid * SLOTS
            pltpu.sync_copy(idx_hbm.at[pl.ds(base, SLOTS)], idx_v)
            pltpu.async_copy(x_hbm.at[idx_v], rows_v, sem).wait()
            pltpu.sync_copy(rows_v, xs_hbm.at[pl.ds(base, SLOTS)])

    RT = 64
    NR = N // RT
    WR = RT + 16
    ENHR = K + N // 16

    @functools.partial(
        pl.kernel, mesh=mesh,
        compiler_params=pltpu.CompilerParams(needs_layout_passes=False),
        out_type=jax.ShapeDtypeStruct((B * NP, CP), F32),
        scratch_types=[pltpu.VMEM((RT,), F32),
                       pltpu.VMEM((NT,), F32),
                       pltpu.VMEM((WR,), I32),
                       pltpu.VMEM((RT, CP), F32),
                       pltpu.VMEM((WR, CP), F32),
                       pltpu.SemaphoreType.DMA],
    )
    def scatter(x_hbm, selm_hbm, bounds_hbm, enh_hbm, out_hbm,
                maskv, bv, destv, xbuf, ebuf, sem):
        iota16 = lax.iota(I32, 16)
        wid = lax.axis_index("s") * NC + lax.axis_index("c")
        for b in range(B):
            pltpu.sync_copy(bounds_hbm.at[pl.ds(b * NT, NT)], bv)
            for gi in range(NR // NW):
                g = wid + gi * NW
                tok0 = b * N + g * RT
                pltpu.sync_copy(x_hbm.at[pl.ds(tok0, RT), :], xbuf)
                pltpu.sync_copy(xbuf,
                                out_hbm.at[pl.ds(b * NP + g * RT, RT), :])
                lo = I32(0)
                for v in range(NT // 16):
                    hit = (1 - jnp.minimum(jnp.abs(iota16 + v * 16 - g), 1))
                    lo = lo + jnp.sum(bv[pl.ds(v * 16, 16)].astype(I32)
                                      * hit)
                lo_al = (lo // 8) * 8
                sh = lo - lo_al
                pltpu.sync_copy(selm_hbm.at[pl.ds(tok0, RT)], maskv)
                dump = b * NP + N + g
                for cvec in range(WR // 16):
                    destv[pl.ds(cvec * 16, 16)] = jnp.full((16,), dump, I32)

                def body(cvec, cursor):
                    mi = maskv[pl.ds(cvec * 16, 16)].astype(I32)
                    vals = (mi * (iota16 + (cvec * 16 + b * NP + g * RT))
                            + (1 - mi) * dump)
                    pos = (mi * (sh + cursor + plsc.cumsum(mi) - 1)
                           + (1 - mi) * I32(WR - 1))
                    plsc.store_scatter(destv, [pos], vals)
                    return cursor + jnp.sum(mi)

                jax.lax.fori_loop(0, RT // 16, body, I32(0))
                pltpu.sync_copy(
                    enh_hbm.at[pl.ds(b * ENHR + lo_al, WR), :], ebuf)
                pltpu.async_copy(ebuf, out_hbm.at[destv], sem).wait()

    return compact, gather, scatter


def kernel(x, boundary_map, w_imp1, b_imp1, w_imp2, b_imp2,
           Wq, bq, Wk, bk, Wv, bv, Wo, bo, ln_g, ln_b):
    B, C, H, W = x.shape
    N = H * W
    K = max(int(N * 0.25), 1)
    heads = 8
    hd = C // heads
    NT = N // 64
    NP = N + 64

    x_t = x.reshape(B, C, N)
    CP = 128
    x2d = jnp.pad(jnp.transpose(x_t, (0, 2, 1)).reshape(B * N, C),
                  ((0, 0), (0, CP - C)))
    bnd = boundary_map.reshape(B, 1, N)

    full = lambda s: pl.BlockSpec(s, lambda b: (0,) * len(s))
    imp, selm, bounds = pl.pallas_call(
        functools.partial(_sel_kernel, N=N, K=K),
        grid=(B,),
        in_specs=[
            pl.BlockSpec((1, C, N), lambda b: (b, 0, 0)),
            pl.BlockSpec((1, 1, N), lambda b: (b, 0, 0)),
            full((C // 4, C)), full((C // 4, 1)),
            full((1, C // 4)), full((1, 1)),
        ],
        out_specs=[
            pl.BlockSpec((1, 1, N), lambda b: (b, 0, 0)),
            pl.BlockSpec((1, 1, N), lambda b: (b, 0, 0)),
            pl.BlockSpec((1, NT // 2, 2), lambda b: (b, 0, 0)),
        ],
        out_shape=[
            jax.ShapeDtypeStruct((B, 1, N), F32),
            jax.ShapeDtypeStruct((B, 1, N), F32),
            jax.ShapeDtypeStruct((B, NT // 2, 2), F32),
        ],
        compiler_params=pltpu.CompilerParams(
            dimension_semantics=("parallel",)),
    )(x_t, bnd, w_imp1, b_imp1.reshape(-1, 1), w_imp2, b_imp2.reshape(1, 1))

    compact, gather, scatter = _make_sc_kernels(B, N, K, C, NT)
    selm1d = selm.reshape(B * N)
    idx = compact(selm1d)
    xs2d = gather(x2d, idx)

    enh = pl.pallas_call(
        functools.partial(_attn_kernel, K=K, heads=heads, hd=hd),
        grid=(B,),
        in_specs=[
            pl.BlockSpec((1, K, CP), lambda b: (b, 0, 0)),
            full((C, C)), full((1, C)),
            full((C, C)), full((1, C)),
            full((C, C)), full((1, C)),
            full((C, C)), full((1, C)),
            full((1, C)), full((1, C)),
        ],
        out_specs=[pl.BlockSpec((1, K + N // 16, CP), lambda b: (b, 0, 0))],
        out_shape=[jax.ShapeDtypeStruct((B, K + N // 16, CP), F32)],
        compiler_params=pltpu.CompilerParams(
            dimension_semantics=("parallel",)),
        scratch_shapes=[pltpu.VMEM((K, C), F32)] * 4,
    )(xs2d.reshape(B, K, CP),
      Wq, bq.reshape(1, -1), Wk, bk.reshape(1, -1), Wv, bv.reshape(1, -1),
      Wo, bo.reshape(1, -1), ln_g.reshape(1, -1), ln_b.reshape(1, -1))[0]

    out2d = scatter(x2d, selm1d, bounds.reshape(B * NT),
                    enh.reshape(B * (K + N // 16), CP))
    out_flat = out2d.reshape(B, NP, CP)[:, :N, :C]
    out = jnp.transpose(out_flat.reshape(B, H, W, C), (0, 3, 1, 2))
    importance = imp.reshape(B, 1, H, W)
    return (out, importance)


# speedup vs baseline: 5.3502x; 1.5219x over previous
import functools

import jax
import jax.numpy as jnp
from jax.experimental import pallas as pl
from jax.experimental.pallas import tpu as pltpu

HIGHEST = jax.lax.Precision.HIGHEST


def _mm(a, b, dims, prec=jax.lax.Precision.DEFAULT):
    return jax.lax.dot_general(a, b, (dims, ((), ())), precision=prec)


def _dsa_kernel(xt_ref, x_ref, bnd_ref, w1_ref, b1_ref, w2_ref, b2_ref,
                wq_ref, bq_ref, wk_ref, bk_ref, wv_ref, bv_ref,
                wo_ref, bo_ref, lng_ref, lnb_ref,
                out_ref, imp_ref,
                sel_ref, rank_ref, xs_ref, q_ref, k_ref, v_ref, ctx_ref,
                *, N, K, heads, hd):
    f32 = jnp.float32
    C = x_ref.shape[2]
    scale = f32(hd) ** -0.5

    h1t = _mm(w1_ref[...], xt_ref[0], ((1,), (0,)), HIGHEST) + b1_ref[...]
    h1t = h1t * f32(0.5) * (f32(1.0) + jax.lax.erf(h1t * f32(0.7071067811865476)))
    logit = _mm(w2_ref[...], h1t, ((1,), (0,)), HIGHEST) + b2_ref[...]
    imp = jax.nn.sigmoid(logit) + f32(0.5) * bnd_ref[0]
    imp_ref[0] = imp

    bits = jax.lax.bitcast_convert_type(imp, jnp.int32)

    def cnt_ge(thr):
        return jnp.sum((bits >= thr).astype(jnp.int32))

    def t_body(i, t):
        cand = t | (jnp.int32(1) << (jnp.int32(30) - i))
        return jnp.where(cnt_ge(cand) >= K, cand, t)

    t = jax.lax.fori_loop(0, 31, t_body, jnp.int32(0))
    need = K - cnt_ge(t + 1)

    idx_row = jax.lax.broadcasted_iota(jnp.int32, (1, N), 1)
    key = jnp.where(bits == t, jnp.int32(N - 1) - idx_row, jnp.int32(-1))

    def th_body(i, th):
        cand = th | (jnp.int32(1) << (jnp.int32(11) - i))
        cnt = jnp.sum((key >= cand).astype(jnp.int32))
        return jnp.where(cnt >= need, cand, th)

    th2 = jax.lax.fori_loop(0, 12, th_body, jnp.int32(0))
    sel_row = ((bits > t) | (key >= th2)).astype(f32)

    sel32 = sel_row.reshape(N // 128, 128)
    lane_i = jax.lax.broadcasted_iota(jnp.int32, (128, 128), 0)
    lane_j = jax.lax.broadcasted_iota(jnp.int32, (128, 128), 1)
    Ustrict = (lane_i < lane_j).astype(f32)
    row_i = jax.lax.broadcasted_iota(jnp.int32, (N // 128, N // 128), 0)
    row_j = jax.lax.broadcasted_iota(jnp.int32, (N // 128, N // 128), 1)
    Lstrict = (row_j < row_i).astype(f32)
    prefix_in = _mm(sel32, Ustrict, ((1,), (0,)))
    rowsum = jnp.sum(sel32, axis=1, keepdims=True)
    offs = _mm(Lstrict, rowsum, ((1,), (0,)))
    rank32 = prefix_in + offs
    rank_row = rank32.reshape(1, N)
    sel_ref[...] = jnp.swapaxes(sel_row, 0, 1)
    rank_ref[...] = jnp.swapaxes(rank_row, 0, 1)

    GCH = 256
    rho0 = jax.lax.broadcasted_iota(jnp.int32, (GCH, N), 0)
    rank_i = rank_row.astype(jnp.int32)

    def gather_body(c, carry):
        G = ((rho0 == rank_i - c * GCH) & (sel_row > f32(0.5))).astype(f32)
        xs_ref[pl.ds(c * GCH, GCH), :] = _mm(G, x_ref[0], ((1,), (0,)))
        return carry

    jax.lax.fori_loop(0, K // GCH, gather_body, 0)

    def lin(a, w_ref, b_ref):
        return _mm(a, w_ref[...], ((1,), (1,))) + b_ref[...]

    xs = xs_ref[...]
    q_ref[...] = lin(xs, wq_ref, bq_ref)
    k_ref[...] = lin(xs, wk_ref, bk_ref)
    v_ref[...] = lin(xs, wv_ref, bv_ref)
    ctx_ref[...] = jnp.zeros((K, C), dtype=f32)

    col = jax.lax.broadcasted_iota(jnp.int32, (1, C), 1)

    def head_body(h, carry):
        hm = ((col >= h * hd) & (col < (h + 1) * hd)).astype(f32)
        e = jnp.exp(_mm(q_ref[...] * hm, k_ref[...], ((1,), (1,))) * scale)
        ctx = _mm(e, v_ref[...] * hm, ((1,), (0,)))
        denom = jnp.sum(e, axis=1, keepdims=True)
        ctx_ref[...] += ctx / denom
        return carry

    jax.lax.fori_loop(0, heads, head_body, 0)

    y = lin(ctx_ref[...], wo_ref, bo_ref) + xs
    mu = jnp.mean(y, axis=1, keepdims=True)
    var = jnp.mean((y - mu) ** 2, axis=1, keepdims=True)
    enh = (y - mu) * jax.lax.rsqrt(var + f32(1e-5)) * lng_ref[...] + lnb_ref[...]
    q_ref[...] = enh

    SCH = 512
    rho1 = jax.lax.broadcasted_iota(jnp.int32, (SCH, K), 1)

    def scatter_body(c, carry):
        rk = rank_ref[pl.ds(c * SCH, SCH), :].astype(jnp.int32)
        sl = sel_ref[pl.ds(c * SCH, SCH), :]
        H = ((rho1 == rk) & (sl > f32(0.5))).astype(f32)
        scat = _mm(H, q_ref[...], ((1,), (0,)))
        xw = x_ref[0, pl.ds(c * SCH, SCH), :]
        out_ref[0, pl.ds(c * SCH, SCH), :] = jnp.where(sl > f32(0.5), scat, xw)
        return carry

    jax.lax.fori_loop(0, N // SCH, scatter_body, 0)


def kernel(x, boundary_map, w_imp1, b_imp1, w_imp2, b_imp2,
           Wq, bq, Wk, bk, Wv, bv, Wo, bo, ln_g, ln_b):
    B, C, H, W = x.shape
    N = H * W
    K = max(int(N * 0.25), 1)
    heads = 8
    hd = C // heads

    x_t = x.reshape(B, C, N)
    x_flat = jnp.transpose(x_t, (0, 2, 1))
    bnd = boundary_map.reshape(B, 1, N)

    full = lambda s: pl.BlockSpec(s, lambda b: (0,) * len(s))
    out_flat, imp = pl.pallas_call(
        functools.partial(_dsa_kernel, N=N, K=K, heads=heads, hd=hd),
        grid=(B,),
        in_specs=[
            pl.BlockSpec((1, C, N), lambda b: (b, 0, 0)),
            pl.BlockSpec((1, N, C), lambda b: (b, 0, 0)),
            pl.BlockSpec((1, 1, N), lambda b: (b, 0, 0)),
            full((C // 4, C)), full((C // 4, 1)),
            full((1, C // 4)), full((1, 1)),
            full((C, C)), full((1, C)),
            full((C, C)), full((1, C)),
            full((C, C)), full((1, C)),
            full((C, C)), full((1, C)),
            full((1, C)), full((1, C)),
        ],
        out_specs=[
            pl.BlockSpec((1, N, C), lambda b: (b, 0, 0)),
            pl.BlockSpec((1, 1, N), lambda b: (b, 0, 0)),
        ],
        out_shape=[
            jax.ShapeDtypeStruct((B, N, C), jnp.float32),
            jax.ShapeDtypeStruct((B, 1, N), jnp.float32),
        ],
        compiler_params=pltpu.CompilerParams(
            dimension_semantics=("parallel",)),
        scratch_shapes=[
            pltpu.VMEM((N, 1), jnp.float32),
            pltpu.VMEM((N, 1), jnp.float32),
            pltpu.VMEM((K, C), jnp.float32),
            pltpu.VMEM((K, C), jnp.float32),
            pltpu.VMEM((K, C), jnp.float32),
            pltpu.VMEM((K, C), jnp.float32),
            pltpu.VMEM((K, C), jnp.float32),
        ],
    )(x_t, x_flat, bnd,
      w_imp1, b_imp1.reshape(-1, 1), w_imp2, b_imp2.reshape(1, 1),
      Wq, bq.reshape(1, -1), Wk, bk.reshape(1, -1), Wv, bv.reshape(1, -1),
      Wo, bo.reshape(1, -1), ln_g.reshape(1, -1), ln_b.reshape(1, -1))

    out = jnp.transpose(out_flat.reshape(B, H, W, C), (0, 3, 1, 2))
    importance = imp.reshape(B, 1, H, W)
    return (out, importance)
